# baseline (speedup 1.0000x reference)
import jax
import jax.numpy as jnp
from jax import lax
from jax.experimental import pallas as pl
from jax.experimental.pallas import tpu as pltpu
from jax.experimental.pallas import tpu_sc as plsc

BATCH = 16384
DIM = 32
NBOUND = 1000
NBPAD = 1024
NC = 2
NS = 16
NW = NC * NS
BPW = BATCH // NW
L = 16
OUTW = 128


def _body(uid_hbm, ts_hbm, utab_hbm, ttab_hbm, bnd_hbm, mean_hbm, istd_hbm,
          out_hbm,
          idx_v, ts_v, bnds_v, mean_v, istd_v, out_v, usem, tsem):
    c = lax.axis_index("c")
    s = lax.axis_index("s")
    wid = s * NC + c
    base = wid * BPW

    def fire_row(tab_hbm, row_id, r, col0, sem):
        off = pl.multiple_of(row_id * DIM, 8)
        pltpu.async_copy(tab_hbm.at[pl.ds(off, DIM)],
                         out_v.at[r, pl.ds(col0, DIM)], sem)

    pltpu.sync_copy(uid_hbm.at[wid], idx_v)

    def ufire(g, carry):
        vec = idx_v[0, pl.ds(g * L, L)]
        for b in range(L):
            fire_row(utab_hbm, vec[b], g * L + b, 0, usem)
        return carry
    lax.fori_loop(0, BPW // L, ufire, 0)

    pltpu.sync_copy(ts_hbm.at[wid], ts_v)
    pltpu.sync_copy(bnd_hbm, bnds_v)
    pltpu.sync_copy(mean_hbm, mean_v)
    pltpu.sync_copy(istd_hbm, istd_v)
    mean = mean_v[...]
    istd = istd_v[...]

    norm_col = jnp.full((L,), 2 * DIM, jnp.int32)
    for p in range(BPW // L):
        tf = ts_v[0, pl.ds(p * L, L)].astype(jnp.float32)
        pos = jnp.zeros((L,), jnp.int32)
        for w in (512, 256, 128, 64, 32, 16, 8, 4, 2, 1):
            probe = plsc.load_gather(bnds_v, [pos + (w - 1)])
            pos = jnp.where(probe <= tf, pos + w, pos)
        for b in range(L):
            fire_row(ttab_hbm, pos[b], p * L + b, DIM, tsem)
        rows = p * L + jnp.arange(L, dtype=jnp.int32)
        plsc.store_scatter(out_v, [rows, norm_col], (tf - mean) * istd)

    pltpu.make_async_copy(out_hbm.at[pl.ds(0, BPW // 4)],
                          out_v.at[pl.ds(0, BPW // 4), :], usem).wait()
    pltpu.make_async_copy(out_hbm.at[pl.ds(0, BPW // 4)],
                          out_v.at[pl.ds(0, BPW // 4), :], tsem).wait()

    pltpu.sync_copy(out_v, out_hbm.at[pl.ds(base, BPW)])


def kernel(userID, review_date_in_unix, user_table, ts_table, boundaries,
           ts_mean, ts_std):
    uid = userID.reshape(NW, 1, BPW)
    ts = review_date_in_unix.reshape(NW, 1, BPW)
    utab = user_table.T.reshape(4, 8, -1).transpose(2, 0, 1).reshape(-1)
    ttab = ts_table.reshape(-1)
    bpad = jnp.concatenate([
        boundaries.astype(jnp.float32),
        jnp.full((NBPAD - NBOUND,), jnp.inf, jnp.float32),
    ])
    mean_v = jnp.broadcast_to(ts_mean.astype(jnp.float32), (L,))
    istd_v = jnp.broadcast_to((1.0 / ts_std).astype(jnp.float32), (L,))

    mesh = plsc.VectorSubcoreMesh(core_axis_name="c", subcore_axis_name="s")
    run = pl.kernel(
        _body,
        out_type=jax.ShapeDtypeStruct((BATCH, OUTW), jnp.float32),
        mesh=mesh,
        scratch_types=[
            pltpu.VMEM((1, BPW), jnp.int32),
            pltpu.VMEM((1, BPW), jnp.int32),
            pltpu.VMEM((NBPAD,), jnp.float32),
            pltpu.VMEM((L,), jnp.float32),
            pltpu.VMEM((L,), jnp.float32),
            pltpu.VMEM((BPW, OUTW), jnp.float32),
            pltpu.SemaphoreType.DMA,
            pltpu.SemaphoreType.DMA,
        ],
        compiler_params=pltpu.CompilerParams(needs_layout_passes=False),
    )
    out = run(uid, ts, utab, ttab, bpad, mean_v, istd_v)
    return out[:, : 2 * DIM + 1]


# speedup vs baseline: 5.4252x; 5.4252x over previous
import jax
import jax.numpy as jnp
from jax import lax
from jax.experimental import pallas as pl
from jax.experimental.pallas import tpu as pltpu
from jax.experimental.pallas import tpu_sc as plsc

BATCH = 16384
DIM = 32
NBOUND = 1000
NBPAD = 1024
NC = 2
NS = 16
NW = NC * NS
BPW = BATCH // NW
L = 16
OUTW = 128
NBUF = 16
TSROWS = NBOUND + 8


def _body(uid_hbm, ts_hbm, utab_hbm, ttab_hbm, bnd_hbm, mean_hbm, istd_hbm,
          out_hbm,
          idx_v, bkt_v, ts_v, bnds_v, mean_v, istd_v,
          out_v, ring, *sems):
    c = lax.axis_index("c")
    s = lax.axis_index("s")
    wid = s * NC + c
    base = wid * BPW

    pltpu.sync_copy(uid_hbm.at[wid], idx_v)

    def make_phase(tab_hbm, idx_ref, col0):
        def idx_vec(i0):
            return idx_ref[0, pl.ds(i0, L)]

        def fire(row, b):
            off = pl.multiple_of((row >> 3) * 8, 8)
            pltpu.async_copy(tab_hbm.at[pl.ds(off, 8)], ring.at[b], sems[b])

        def drain(b):
            pltpu.make_async_copy(tab_hbm.at[pl.ds(0, 8)], ring.at[b],
                                  sems[b]).wait()

        def consume(i, sub, b):
            out_v[i, pl.ds(col0, L)] = ring[b, sub, pl.ds(0, L)]
            out_v[i, pl.ds(col0 + L, L)] = ring[b, sub, pl.ds(L, L)]

        return idx_vec, fire, drain, consume

    uvec, ufire, udrain, uconsume = make_phase(utab_hbm, idx_v, 0)
    tvec, tfire, tdrain, tconsume = make_phase(ttab_hbm, bkt_v, DIM)

    vec0 = uvec(0)
    for b in range(NBUF):
        ufire(vec0[b], b)

    pltpu.sync_copy(ts_hbm.at[wid], ts_v)
    pltpu.sync_copy(bnd_hbm, bnds_v)
    pltpu.sync_copy(mean_hbm, mean_v)
    pltpu.sync_copy(istd_hbm, istd_v)
    mean = mean_v[...]
    istd = istd_v[...]

    norm_col = jnp.full((L,), 2 * DIM, jnp.int32)
    for p in range(BPW // L):
        tf = ts_v[0, pl.ds(p * L, L)].astype(jnp.float32)
        pos = jnp.zeros((L,), jnp.int32)
        for w in (512, 256, 128, 64, 32, 16, 8, 4, 2, 1):
            probe = plsc.load_gather(bnds_v, [pos + (w - 1)])
            pos = jnp.where(probe <= tf, pos + w, pos)
        bkt_v[0, pl.ds(p * L, L)] = pos
        rows = p * L + jnp.arange(L, dtype=jnp.int32)
        plsc.store_scatter(out_v, [rows, norm_col], (tf - mean) * istd)

    nblk = BPW // NBUF

    def ubody(g, carry):
        i0 = g * NBUF
        cur = uvec(i0)
        nxt = uvec(i0 + NBUF)
        for b in range(NBUF):
            udrain(b)
            uconsume(i0 + b, cur[b] & 7, b)
            ufire(nxt[b], b)
        return carry
    lax.fori_loop(0, nblk - 1, ubody, 0)
    lastu = uvec((nblk - 1) * NBUF)
    tvec0 = tvec(0)
    for b in range(NBUF):
        udrain(b)
        uconsume((nblk - 1) * NBUF + b, lastu[b] & 7, b)
        tfire(tvec0[b], b)

    def tbody(g, carry):
        i0 = g * NBUF
        cur = tvec(i0)
        nxt = tvec(i0 + NBUF)
        for b in range(NBUF):
            tdrain(b)
            tconsume(i0 + b, cur[b] & 7, b)
            tfire(nxt[b], b)
        return carry
    lax.fori_loop(0, nblk - 1, tbody, 0)
    lastt = tvec((nblk - 1) * NBUF)
    for b in range(NBUF):
        tdrain(b)
        tconsume((nblk - 1) * NBUF + b, lastt[b] & 7, b)

    pltpu.sync_copy(out_v, out_hbm.at[pl.ds(base, BPW)])


def kernel(userID, review_date_in_unix, user_table, ts_table, boundaries,
           ts_mean, ts_std):
    uid = userID.reshape(NW, 1, BPW)
    ts = review_date_in_unix.reshape(NW, 1, BPW)
    ttab = jnp.pad(ts_table, ((0, TSROWS - (NBOUND + 1)), (0, 0)))
    bpad = jnp.concatenate([
        boundaries.astype(jnp.float32),
        jnp.full((NBPAD - NBOUND,), jnp.inf, jnp.float32),
    ])
    mean_v = jnp.broadcast_to(ts_mean.astype(jnp.float32), (L,))
    istd_v = jnp.broadcast_to((1.0 / ts_std).astype(jnp.float32), (L,))

    mesh = plsc.VectorSubcoreMesh(core_axis_name="c", subcore_axis_name="s")
    run = pl.kernel(
        _body,
        out_type=jax.ShapeDtypeStruct((BATCH, OUTW), jnp.float32),
        mesh=mesh,
        scratch_types=[
            pltpu.VMEM((1, BPW), jnp.int32),
            pltpu.VMEM((1, BPW), jnp.int32),
            pltpu.VMEM((1, BPW), jnp.int32),
            pltpu.VMEM((NBPAD,), jnp.float32),
            pltpu.VMEM((L,), jnp.float32),
            pltpu.VMEM((L,), jnp.float32),
            pltpu.VMEM((BPW, OUTW), jnp.float32),
            pltpu.VMEM((NBUF, 8, DIM), jnp.float32),
        ] + [pltpu.SemaphoreType.DMA] * NBUF,
        compiler_params=pltpu.CompilerParams(needs_layout_passes=False),
    )
    out = run(uid, ts, user_table, ttab, bpad, mean_v, istd_v)
    return out[:, : 2 * DIM + 1]


# speedup vs baseline: 5.6657x; 1.0443x over previous
import jax
import jax.numpy as jnp
from jax import lax
from jax.experimental import pallas as pl
from jax.experimental.pallas import tpu as pltpu
from jax.experimental.pallas import tpu_sc as plsc

BATCH = 16384
DIM = 32
NBOUND = 1000
NBPAD = 1024
NC = 2
NS = 16
NW = NC * NS
BPW = BATCH // NW
L = 16
OUTW = 128
NBUF = 16
TSROWS = NBOUND + 8


def _body(uid_hbm, ts_hbm, utab_hbm, ttab_hbm, bnd_hbm, mean_hbm, istd_hbm,
          out_hbm,
          idx_v, bkt_v, ts_v, bnds_v, mean_v, istd_v,
          out_v, ring, *sems):
    c = lax.axis_index("c")
    s = lax.axis_index("s")
    wid = s * NC + c
    base = wid * BPW

    pltpu.sync_copy(uid_hbm.at[wid], idx_v)

    def make_phase(tab_hbm, idx_ref, col0):
        def idx_vec(i0):
            return idx_ref[0, pl.ds(i0, L)]

        def fire(row, b):
            off = pl.multiple_of((row >> 3) * 8, 8)
            pltpu.async_copy(tab_hbm.at[pl.ds(off, 8)], ring.at[b], sems[b])

        def drain(b):
            pltpu.make_async_copy(tab_hbm.at[pl.ds(0, 8)], ring.at[b],
                                  sems[b]).wait()

        def consume(i, sub, b):
            out_v[i, pl.ds(col0, L)] = ring[b, sub, pl.ds(0, L)]
            out_v[i, pl.ds(col0 + L, L)] = ring[b, sub, pl.ds(L, L)]

        return idx_vec, fire, drain, consume

    uvec, ufire, udrain, uconsume = make_phase(utab_hbm, idx_v, 0)
    tvec, tfire, tdrain, tconsume = make_phase(ttab_hbm, bkt_v, DIM)

    vec0 = uvec(0)
    for b in range(NBUF // 2):
        ufire(vec0[b], b)

    pltpu.sync_copy(ts_hbm.at[wid], ts_v)
    pltpu.sync_copy(bnd_hbm, bnds_v)
    pltpu.sync_copy(mean_hbm, mean_v)
    pltpu.sync_copy(istd_hbm, istd_v)
    mean = mean_v[...]
    istd = istd_v[...]

    norm_col = jnp.full((L,), 2 * DIM, jnp.int32)
    for p in range(BPW // L):
        tf = ts_v[0, pl.ds(p * L, L)].astype(jnp.float32)
        pos = jnp.zeros((L,), jnp.int32)
        for w in (512, 256, 128, 64, 32, 16, 8, 4, 2, 1):
            probe = plsc.load_gather(bnds_v, [pos + (w - 1)])
            pos = jnp.where(probe <= tf, pos + w, pos)
        bkt_v[0, pl.ds(p * L, L)] = pos
        rows = p * L + jnp.arange(L, dtype=jnp.int32)
        plsc.store_scatter(out_v, [rows, norm_col], (tf - mean) * istd)

    nblk = BPW // NBUF
    H = NBUF // 2

    def ubody(g, carry):
        i0 = g * NBUF
        cur = uvec(i0)
        nxt = uvec(i0 + NBUF)
        for b in range(NBUF):
            udrain(b)
            uconsume(i0 + b, cur[b] & 7, b)
            nrow = cur[b + H] if b < H else nxt[b - H]
            ufire(nrow, (b + H) % NBUF)
        return carry
    lax.fori_loop(0, nblk - 1, ubody, 0)
    lastu = uvec((nblk - 1) * NBUF)
    tvec0 = tvec(0)
    for b in range(NBUF):
        udrain(b)
        uconsume((nblk - 1) * NBUF + b, lastu[b] & 7, b)
        nrow = lastu[b + H] if b < H else tvec0[b - H]
        if b < H:
            ufire(nrow, (b + H) % NBUF)
        else:
            tfire(nrow, (b + H) % NBUF)

    def tbody(g, carry):
        i0 = g * NBUF
        cur = tvec(i0)
        nxt = tvec(i0 + NBUF)
        for b in range(NBUF):
            tdrain(b)
            tconsume(i0 + b, cur[b] & 7, b)
            nrow = cur[b + H] if b < H else nxt[b - H]
            tfire(nrow, (b + H) % NBUF)
        return carry
    lax.fori_loop(0, nblk - 1, tbody, 0)
    lastt = tvec((nblk - 1) * NBUF)
    for b in range(NBUF):
        tdrain(b)
        tconsume((nblk - 1) * NBUF + b, lastt[b] & 7, b)
        if b < H:
            tfire(lastt[b + H], (b + H) % NBUF)

    pltpu.sync_copy(out_v, out_hbm.at[pl.ds(base, BPW)])


def kernel(userID, review_date_in_unix, user_table, ts_table, boundaries,
           ts_mean, ts_std):
    uid = userID.reshape(NW, 1, BPW)
    ts = review_date_in_unix.reshape(NW, 1, BPW)
    ttab = jnp.pad(ts_table, ((0, TSROWS - (NBOUND + 1)), (0, 0)))
    bpad = jnp.concatenate([
        boundaries.astype(jnp.float32),
        jnp.full((NBPAD - NBOUND,), jnp.inf, jnp.float32),
    ])
    mean_v = jnp.broadcast_to(ts_mean.astype(jnp.float32), (L,))
    istd_v = jnp.broadcast_to((1.0 / ts_std).astype(jnp.float32), (L,))

    mesh = plsc.VectorSubcoreMesh(core_axis_name="c", subcore_axis_name="s")
    run = pl.kernel(
        _body,
        out_type=jax.ShapeDtypeStruct((BATCH, OUTW), jnp.float32),
        mesh=mesh,
        scratch_types=[
            pltpu.VMEM((1, BPW), jnp.int32),
            pltpu.VMEM((1, BPW), jnp.int32),
            pltpu.VMEM((1, BPW), jnp.int32),
            pltpu.VMEM((NBPAD,), jnp.float32),
            pltpu.VMEM((L,), jnp.float32),
            pltpu.VMEM((L,), jnp.float32),
            pltpu.VMEM((BPW, OUTW), jnp.float32),
            pltpu.VMEM((NBUF, 8, DIM), jnp.float32),
        ] + [pltpu.SemaphoreType.DMA] * NBUF,
        compiler_params=pltpu.CompilerParams(needs_layout_passes=False),
    )
    out = run(uid, ts, user_table, ttab, bpad, mean_v, istd_v)
    return out[:, : 2 * DIM + 1]
